# Initial kernel scaffold; baseline (speedup 1.0000x reference)
import jax
import jax.numpy as jnp
from jax.experimental import pallas as pl


def kernel(positions, atom_types):
    raise NotImplementedError("write your pallas kernel here")


# speedup vs baseline: 1.3696x; 1.3696x over previous
import jax
import jax.numpy as jnp
from jax.experimental import pallas as pl

_RCUT = 6.0
_NMAX = 8
_LMAX = 6
_SIGMA = 0.5
_N = 2048
_OUT_COLS = _NMAX * _N + (_LMAX + 1)
_BR = 128


def _soap_block(pos_row_ref, pos_t_ref, out_ref):
    xi = pos_row_ref[:, 0:1]
    yi = pos_row_ref[:, 1:2]
    zi = pos_row_ref[:, 2:3]
    dx = xi - pos_t_ref[0:1, :]
    dy = yi - pos_t_ref[1:2, :]
    dz = zi - pos_t_ref[2:3, :]
    d2 = dx * dx + dy * dy + dz * dz + 1e-10
    dist = jnp.sqrt(d2)
    alpha = 0.5 / (_SIGMA * _SIGMA)
    env = jnp.exp(-alpha * (dist * dist))
    mask = (dist < _RCUT) & (dist > 0.1)
    g = jnp.where(mask, env, 0.0)
    r = dist * (1.0 / _RCUT)
    out_ref[:, 0:_N] = g
    for n in range(1, _NMAX):
        g = g * r
        out_ref[:, n * _N:(n + 1) * _N] = g
    out_ref[:, _NMAX * _N:] = jnp.zeros((_BR, _LMAX + 1), jnp.float32)


def kernel(positions, atom_types):
    del atom_types
    pos_t = positions.T
    grid = (_N // _BR,)
    return pl.pallas_call(
        _soap_block,
        grid=grid,
        in_specs=[
            pl.BlockSpec((_BR, 3), lambda i: (i, 0)),
            pl.BlockSpec((3, _N), lambda i: (0, 0)),
        ],
        out_specs=pl.BlockSpec((_BR, _OUT_COLS), lambda i: (i, 0)),
        out_shape=jax.ShapeDtypeStruct((_N, _OUT_COLS), jnp.float32),
    )(positions, pos_t)


# speedup vs baseline: 1.3714x; 1.0013x over previous
import jax
import jax.numpy as jnp
from jax.experimental import pallas as pl
from jax.experimental.pallas import tpu as pltpu

_RCUT = 6.0
_NMAX = 8
_LMAX = 6
_SIGMA = 0.5
_N = 2048
_OUT_COLS = _NMAX * _N + (_LMAX + 1)
_BR = 128


def _soap_block(pos_row_ref, pos_t_ref, out_ref):
    xi = pos_row_ref[:, 0:1]
    yi = pos_row_ref[:, 1:2]
    zi = pos_row_ref[:, 2:3]
    dx = xi - pos_t_ref[0:1, :]
    dy = yi - pos_t_ref[1:2, :]
    dz = zi - pos_t_ref[2:3, :]
    d2 = dx * dx + dy * dy + dz * dz + 1e-10
    dist = jnp.sqrt(d2)
    alpha = 0.5 / (_SIGMA * _SIGMA)
    env = jnp.exp(-alpha * (dist * dist))
    mask = (dist < _RCUT) & (dist > 0.1)
    g = jnp.where(mask, env, 0.0)
    r = dist * (1.0 / _RCUT)
    out_ref[:, 0:_N] = g
    for n in range(1, _NMAX):
        g = g * r
        out_ref[:, n * _N:(n + 1) * _N] = g
    out_ref[:, _NMAX * _N:] = jnp.zeros((_BR, _LMAX + 1), jnp.float32)


def kernel(positions, atom_types):
    del atom_types
    pos_t = positions.T
    grid = (_N // _BR,)
    return pl.pallas_call(
        _soap_block,
        grid=grid,
        in_specs=[
            pl.BlockSpec((_BR, 3), lambda i: (i, 0)),
            pl.BlockSpec((3, _N), lambda i: (0, 0)),
        ],
        out_specs=pl.BlockSpec((_BR, _OUT_COLS), lambda i: (i, 0)),
        out_shape=jax.ShapeDtypeStruct((_N, _OUT_COLS), jnp.float32),
        compiler_params=pltpu.CompilerParams(
            dimension_semantics=("parallel",),
        ),
    )(positions, pos_t)
